# Initial kernel scaffold; baseline (speedup 1.0000x reference)
import jax
import jax.numpy as jnp
from jax.experimental import pallas as pl


def kernel(x, edge_dir, W1, b1, W2, b2, W3, b3, W4, b4, gamma, beta, src_idx, dst_idx):
    raise NotImplementedError("write your pallas kernel here")


# speedup vs baseline: 3.3845x; 3.3845x over previous
import functools

import jax
import jax.numpy as jnp
from jax import lax
from jax.experimental import pallas as pl
from jax.experimental.pallas import tpu as pltpu
from jax.experimental.pallas import tpu_sc as plsc

N = 10000
NP = 10240
E = 320000
H = 128
HP = H + 16
NC = 2
NS = 16
NW = NC * NS
EPT = E // NW
C = 80
NCHT = EPT // C
RPT = NP // NS


def _sc_edges(p_hbm, src_hbm, dst_hbm, ed_hbm, wd_hbm, out_hbm,
              scur0, scur1, dcur0, dcur1, ecur0, ecur1, dscat_v,
              rows0, rows1, orows_v, wd_v, zbuf_v, s_sh,
              sem_g, sem_i, sem_s):
    c = lax.axis_index("c")
    s = lax.axis_index("s")
    wid = c * NS + s
    ebase = wid * EPT

    scur = (scur0, scur1)
    dcur = (dcur0, dcur1)
    ecur = (ecur0, ecur1)
    rows = (rows0, rows1)

    zeros16 = jnp.zeros((16,), jnp.float32)
    ones16 = jnp.ones((16,), jnp.float32)

    for i in range(16):
        for k in range(HP // 16):
            zbuf_v[i, pl.ds(k * 16, 16)] = zeros16

    @pl.loop(0, RPT // 16)
    def _zero_rows(j):
        pltpu.sync_copy(zbuf_v, s_sh.at[pl.ds(s * RPT + j * 16, 16)])

    @pl.loop(0, C)
    def _ones_rows(i):
        orows_v[i, pl.ds(H, 16)] = ones16

    pltpu.sync_copy(wd_hbm, wd_v)
    wd = [wd_v[pl.ds(k * 16, 16)] for k in range(H // 16)]

    def idx_issue(i, p):
        off = i * C
        pltpu.async_copy(src_hbm.at[pl.ds(ebase + off, C)], scur[p], sem_i)
        pltpu.async_copy(dst_hbm.at[pl.ds(ebase + off, C)], dcur[p], sem_i)
        pltpu.async_copy(ed_hbm.at[pl.ds(ebase + off, C)], ecur[p], sem_i)

    def idx_wait(p):
        pltpu.make_async_copy(src_hbm.at[pl.ds(ebase, C)], scur[p], sem_i).wait()
        pltpu.make_async_copy(dst_hbm.at[pl.ds(ebase, C)], dcur[p], sem_i).wait()
        pltpu.make_async_copy(ed_hbm.at[pl.ds(ebase, C)], ecur[p], sem_i).wait()

    def gather_issue(p):
        pltpu.async_copy(p_hbm.at[scur[p]], rows[p], sem_g)

    def gather_wait(p):
        pltpu.make_async_copy(p_hbm.at[scur[p]], rows[p], sem_g).wait()

    def scatter_issue():
        pltpu.async_copy(orows_v, s_sh.at[dscat_v], sem_s, add=True)

    def scatter_wait():
        pltpu.make_async_copy(orows_v, s_sh.at[dscat_v], sem_s).wait()

    dnums = lax.GatherDimensionNumbers(offset_dims=(), collapsed_slice_dims=(0,),
                                       start_index_map=(0,))

    def compute(p):
        rp = rows[p]
        ep = ecur[p]

        @plsc.parallel_loop(0, C // 16)
        def _grp(g):
            D = ep[pl.ds(g * 16, 16)]
            for j in range(16):
                e = g * 16 + j
                lane = jnp.full((16, 1), j, jnp.int32)
                d = lax.gather(D, lane, dnums, slice_sizes=(1,),
                               mode=lax.GatherScatterMode.PROMISE_IN_BOUNDS)
                for k in range(H // 16):
                    sl = pl.ds(k * 16, 16)
                    orows_v[e, sl] = jnp.maximum(rp[e, sl] + d * wd[k], 0.0)

    def dscat_fill(p):
        dp = dcur[p]
        for g in range(C // 16):
            dscat_v[pl.ds(g * 16, 16)] = dp[pl.ds(g * 16, 16)]

    plsc.subcore_barrier()

    idx_issue(0, 0)
    idx_wait(0)
    gather_issue(0)
    idx_issue(1, 1)
    gather_wait(0)
    idx_wait(1)
    gather_issue(1)
    compute(0)
    dscat_fill(0)
    scatter_issue()
    idx_issue(2, 0)

    @pl.loop(0, (NCHT - 1) // 2)
    def _pair(t):
        for b in range(2):
            i = 2 * t + 1 + b
            p = 1 - b
            gather_wait(p)

            @pl.when(i + 1 < NCHT)
            def _():
                idx_wait(1 - p)
                gather_issue(1 - p)

            scatter_wait()
            compute(p)
            dscat_fill(p)
            scatter_issue()

            @pl.when(i + 2 < NCHT)
            def _():
                idx_issue(i + 2, p)

    scatter_wait()

    plsc.subcore_barrier()

    pltpu.sync_copy(s_sh.at[pl.ds(s * RPT, RPT)],
                    out_hbm.at[c, pl.ds(s * RPT, RPT)])


@functools.lru_cache(maxsize=1)
def _sc_edge_call():
    return pl.kernel(
        _sc_edges,
        out_type=jax.ShapeDtypeStruct((NC, NP, HP), jnp.float32),
        mesh=plsc.VectorSubcoreMesh(core_axis_name="c", subcore_axis_name="s",
                                    num_cores=NC, num_subcores=NS),
        compiler_params=pltpu.CompilerParams(use_tc_tiling_on_sc=False,
                                             needs_layout_passes=False),
        scratch_types=[
            pltpu.VMEM((C,), jnp.int32),
            pltpu.VMEM((C,), jnp.int32),
            pltpu.VMEM((C,), jnp.int32),
            pltpu.VMEM((C,), jnp.int32),
            pltpu.VMEM((C,), jnp.float32),
            pltpu.VMEM((C,), jnp.float32),
            pltpu.VMEM((C,), jnp.int32),
            pltpu.VMEM((C, H), jnp.float32),
            pltpu.VMEM((C, H), jnp.float32),
            pltpu.VMEM((C, HP), jnp.float32),
            pltpu.VMEM((H,), jnp.float32),
            pltpu.VMEM((16, HP), jnp.float32),
            pltpu.VMEM_SHARED((NP, HP), jnp.float32),
            pltpu.SemaphoreType.DMA,
            pltpu.SemaphoreType.DMA,
            pltpu.SemaphoreType.DMA,
        ],
    )


def _tc_pre(x_ref, w_ref, b_ref, o_ref):
    o_ref[:] = jnp.dot(x_ref[:], w_ref[:],
                       preferred_element_type=jnp.float32) + b_ref[:]


def _tc_post(x_ref, s_ref, w2, b2, w3a, w3b, b3, w4, b4, g_ref, bt_ref, o_ref):
    S = s_ref[0] + s_ref[1]
    hsum = S[:, :H]
    deg = S[:, H:H + 1]
    agg = jnp.dot(hsum, w2[:], preferred_element_type=jnp.float32) + deg * b2[:]
    u = jnp.maximum(
        jnp.dot(x_ref[:], w3a[:], preferred_element_type=jnp.float32)
        + jnp.dot(agg, w3b[:], preferred_element_type=jnp.float32)
        + b3[:], 0.0)
    upd = jnp.dot(u, w4[:], preferred_element_type=jnp.float32) + b4[:]
    y = x_ref[:] + upd
    mu = jnp.mean(y, axis=-1, keepdims=True)
    var = jnp.mean((y - mu) * (y - mu), axis=-1, keepdims=True)
    o_ref[:] = (y - mu) * lax.rsqrt(var + 1e-5) * g_ref[:] + bt_ref[:]


def kernel(x, edge_dir, W1, b1, W2, b2, W3, b3, W4, b4, gamma, beta,
           src_idx, dst_idx):
    W1a = W1[:H]
    wd = W1[H]
    W3a = W3[:H]
    W3b = W3[H:]

    P = pl.pallas_call(
        _tc_pre,
        out_shape=jax.ShapeDtypeStruct((N, H), jnp.float32),
    )(x, W1a, b1.reshape(1, H))

    Sp = _sc_edge_call()(
        P,
        src_idx,
        dst_idx,
        edge_dir.reshape(E),
        wd,
    )
    S = Sp[:, :N, :]

    out = pl.pallas_call(
        _tc_post,
        out_shape=jax.ShapeDtypeStruct((N, H), jnp.float32),
    )(x, S, W2, b2.reshape(1, H), W3a, W3b, b3.reshape(1, H),
      W4, b4.reshape(1, H), gamma.reshape(1, H), beta.reshape(1, H))
    return out


# speedup vs baseline: 7.9018x; 2.3347x over previous
import functools

import jax
import jax.numpy as jnp
from jax import lax
from jax.experimental import pallas as pl
from jax.experimental.pallas import tpu as pltpu
from jax.experimental.pallas import tpu_sc as plsc

N = 10000
NP = 10240
E = 320000
H = 128
HP = H + 16
NC = 2
NS = 16
NW = NC * NS
EPT = E // NW
C = 80
NCHT = EPT // C
RPT = NP // NS


def _sc_edges(p_hbm, src_hbm, dst_hbm, ed_hbm, wd_hbm, out_hbm,
              scur0, scur1, dcur0, dcur1, ecur0, ecur1, dscat_v,
              rows0, rows1, orows_v, wd_v, zbuf_v, s_sh,
              sem_g, sem_i, sem_s):
    c = lax.axis_index("c")
    s = lax.axis_index("s")
    wid = c * NS + s
    ebase = wid * EPT

    scur = (scur0, scur1)
    dcur = (dcur0, dcur1)
    ecur = (ecur0, ecur1)
    rows = (rows0, rows1)

    zeros16 = jnp.zeros((16,), jnp.float32)
    ones16 = jnp.ones((16,), jnp.float32)

    for i in range(16):
        for k in range(HP // 16):
            zbuf_v[i, pl.ds(k * 16, 16)] = zeros16

    @pl.loop(0, RPT // 16)
    def _zero_rows(j):
        pltpu.sync_copy(zbuf_v, s_sh.at[pl.ds(s * RPT + j * 16, 16)])

    @pl.loop(0, C)
    def _ones_rows(i):
        orows_v[i, pl.ds(H, 16)] = ones16

    pltpu.sync_copy(wd_hbm, wd_v)
    wd = [wd_v[pl.ds(k * 16, 16)] for k in range(H // 16)]

    def idx_issue(i, p):
        off = i * C
        pltpu.async_copy(src_hbm.at[pl.ds(ebase + off, C)], scur[p], sem_i)
        pltpu.async_copy(dst_hbm.at[pl.ds(ebase + off, C)], dcur[p], sem_i)
        pltpu.async_copy(ed_hbm.at[pl.ds(ebase + off, C)], ecur[p], sem_i)

    def idx_wait(p):
        pltpu.make_async_copy(src_hbm.at[pl.ds(ebase, C)], scur[p], sem_i).wait()
        pltpu.make_async_copy(dst_hbm.at[pl.ds(ebase, C)], dcur[p], sem_i).wait()
        pltpu.make_async_copy(ed_hbm.at[pl.ds(ebase, C)], ecur[p], sem_i).wait()

    def gather_issue(p):
        pltpu.async_copy(p_hbm.at[scur[p]], rows[p], sem_g)

    def gather_wait(p):
        pltpu.make_async_copy(p_hbm.at[scur[p]], rows[p], sem_g).wait()

    def scatter_issue():
        pltpu.async_copy(orows_v, s_sh.at[dscat_v], sem_s, add=True)

    def scatter_wait():
        pltpu.make_async_copy(orows_v, s_sh.at[dscat_v], sem_s).wait()

    dnums = lax.GatherDimensionNumbers(offset_dims=(), collapsed_slice_dims=(0,),
                                       start_index_map=(0,))

    def compute(p):
        rp = rows[p]
        ep = ecur[p]

        @plsc.parallel_loop(0, C // 16)
        def _grp(g):
            D = ep[pl.ds(g * 16, 16)]
            for j in range(0, 16, 2):
                es = (g * 16 + j, g * 16 + j + 1)
                ds = []
                for jj in (j, j + 1):
                    lane = jnp.full((16, 1), jj, jnp.int32)
                    ds.append(lax.gather(D, lane, dnums, slice_sizes=(1,),
                                         mode=lax.GatherScatterMode.PROMISE_IN_BOUNDS))
                vals = [[rp[e, pl.ds(k * 16, 16)] for k in range(H // 16)]
                        for e in es]
                outs = [[jnp.maximum(vals[t][k] + ds[t] * wd[k], 0.0)
                         for k in range(H // 16)] for t in range(2)]
                for t in range(2):
                    for k in range(H // 16):
                        orows_v[es[t], pl.ds(k * 16, 16)] = outs[t][k]

    def dscat_fill(p):
        dp = dcur[p]
        for g in range(C // 16):
            dscat_v[pl.ds(g * 16, 16)] = dp[pl.ds(g * 16, 16)]

    plsc.subcore_barrier()

    idx_issue(0, 0)
    idx_wait(0)
    gather_issue(0)
    idx_issue(1, 1)
    gather_wait(0)
    idx_wait(1)
    gather_issue(1)
    compute(0)
    dscat_fill(0)
    scatter_issue()
    idx_issue(2, 0)

    @pl.loop(0, (NCHT - 1) // 2)
    def _pair(t):
        for b in range(2):
            i = 2 * t + 1 + b
            p = 1 - b
            gather_wait(p)

            @pl.when(i + 1 < NCHT)
            def _():
                idx_wait(1 - p)
                gather_issue(1 - p)

            scatter_wait()
            compute(p)
            dscat_fill(p)
            scatter_issue()

            @pl.when(i + 2 < NCHT)
            def _():
                idx_issue(i + 2, p)

    scatter_wait()

    plsc.subcore_barrier()

    pltpu.sync_copy(s_sh.at[pl.ds(s * RPT, RPT)],
                    out_hbm.at[c, pl.ds(s * RPT, RPT)])


@functools.lru_cache(maxsize=1)
def _sc_edge_call():
    return pl.kernel(
        _sc_edges,
        out_type=jax.ShapeDtypeStruct((NC, NP, HP), jnp.float32),
        mesh=plsc.VectorSubcoreMesh(core_axis_name="c", subcore_axis_name="s",
                                    num_cores=NC, num_subcores=NS),
        compiler_params=pltpu.CompilerParams(use_tc_tiling_on_sc=False,
                                             needs_layout_passes=False),
        scratch_types=[
            pltpu.VMEM((C,), jnp.int32),
            pltpu.VMEM((C,), jnp.int32),
            pltpu.VMEM((C,), jnp.int32),
            pltpu.VMEM((C,), jnp.int32),
            pltpu.VMEM((C,), jnp.float32),
            pltpu.VMEM((C,), jnp.float32),
            pltpu.VMEM((C,), jnp.int32),
            pltpu.VMEM((C, H), jnp.float32),
            pltpu.VMEM((C, H), jnp.float32),
            pltpu.VMEM((C, HP), jnp.float32),
            pltpu.VMEM((H,), jnp.float32),
            pltpu.VMEM((16, HP), jnp.float32),
            pltpu.VMEM_SHARED((NP, HP), jnp.float32),
            pltpu.SemaphoreType.DMA,
            pltpu.SemaphoreType.DMA,
            pltpu.SemaphoreType.DMA,
        ],
    )


def _tc_pre(x_ref, w_ref, b_ref, o_ref):
    o_ref[:] = jnp.dot(x_ref[:], w_ref[:],
                       preferred_element_type=jnp.float32) + b_ref[:]


def _tc_post(x_ref, s_ref, w2, b2, w3a, w3b, b3, w4, b4, g_ref, bt_ref, o_ref):
    S = s_ref[0] + s_ref[1]
    hsum = S[:, :H]
    deg = S[:, H:H + 1]
    agg = jnp.dot(hsum, w2[:], preferred_element_type=jnp.float32) + deg * b2[:]
    u = jnp.maximum(
        jnp.dot(x_ref[:], w3a[:], preferred_element_type=jnp.float32)
        + jnp.dot(agg, w3b[:], preferred_element_type=jnp.float32)
        + b3[:], 0.0)
    upd = jnp.dot(u, w4[:], preferred_element_type=jnp.float32) + b4[:]
    y = x_ref[:] + upd
    mu = jnp.mean(y, axis=-1, keepdims=True)
    var = jnp.mean((y - mu) * (y - mu), axis=-1, keepdims=True)
    o_ref[:] = (y - mu) * lax.rsqrt(var + 1e-5) * g_ref[:] + bt_ref[:]


def kernel(x, edge_dir, W1, b1, W2, b2, W3, b3, W4, b4, gamma, beta,
           src_idx, dst_idx):
    W1a = W1[:H]
    wd = W1[H]
    W3a = W3[:H]
    W3b = W3[H:]

    P = pl.pallas_call(
        _tc_pre,
        out_shape=jax.ShapeDtypeStruct((N, H), jnp.float32),
    )(x, W1a, b1.reshape(1, H))

    Sp = _sc_edge_call()(
        P,
        src_idx,
        dst_idx,
        edge_dir.reshape(E),
        wd,
    )
    S = Sp[:, :N, :]

    out = pl.pallas_call(
        _tc_post,
        out_shape=jax.ShapeDtypeStruct((N, H), jnp.float32),
    )(x, S, W2, b2.reshape(1, H), W3a, W3b, b3.reshape(1, H),
      W4, b4.reshape(1, H), gamma.reshape(1, H), beta.reshape(1, H))
    return out


# speedup vs baseline: 8.3460x; 1.0562x over previous
import functools

import jax
import jax.numpy as jnp
from jax import lax
from jax.experimental import pallas as pl
from jax.experimental.pallas import tpu as pltpu
from jax.experimental.pallas import tpu_sc as plsc

N = 10000
NP = 10240
E = 320000
H = 128
HP = H + 16
NC = 2
NS = 16
NW = NC * NS
EPT = E // NW
C = 80
NCHT = EPT // C
RPT = NP // NS


def _sc_edges(p_hbm, src_hbm, dst_hbm, ed_hbm, wd_hbm, out_hbm,
              scur0, scur1, dcur0, dcur1, ecur0, ecur1, dscat_v,
              rows0, rows1, orows_v, wd_v, zbuf_v, s_sh,
              sem_g, sem_i, sem_s):
    c = lax.axis_index("c")
    s = lax.axis_index("s")
    wid = c * NS + s
    ebase = wid * EPT

    scur = (scur0, scur1)
    dcur = (dcur0, dcur1)
    ecur = (ecur0, ecur1)
    rows = (rows0, rows1)

    zeros16 = jnp.zeros((16,), jnp.float32)
    ones16 = jnp.ones((16,), jnp.float32)

    for i in range(16):
        for k in range(HP // 16):
            zbuf_v[i, pl.ds(k * 16, 16)] = zeros16

    @pl.loop(0, RPT // 16)
    def _zero_rows(j):
        pltpu.sync_copy(zbuf_v, s_sh.at[pl.ds(s * RPT + j * 16, 16)])

    @pl.loop(0, C)
    def _ones_rows(i):
        orows_v[i, pl.ds(H, 16)] = ones16

    pltpu.sync_copy(wd_hbm, wd_v)
    wd = [wd_v[pl.ds(k * 16, 16)] for k in range(H // 16)]

    def idx_issue(i, p):
        off = i * C
        pltpu.async_copy(src_hbm.at[pl.ds(ebase + off, C)], scur[p], sem_i)
        pltpu.async_copy(dst_hbm.at[pl.ds(ebase + off, C)], dcur[p], sem_i)
        pltpu.async_copy(ed_hbm.at[pl.ds(ebase + off, C)], ecur[p], sem_i)

    def idx_wait(p):
        pltpu.make_async_copy(src_hbm.at[pl.ds(ebase, C)], scur[p], sem_i).wait()
        pltpu.make_async_copy(dst_hbm.at[pl.ds(ebase, C)], dcur[p], sem_i).wait()
        pltpu.make_async_copy(ed_hbm.at[pl.ds(ebase, C)], ecur[p], sem_i).wait()

    CH = C // 2

    def gather_issue(p):
        pltpu.async_copy(p_hbm.at[scur[p].at[pl.ds(0, CH)]],
                         rows[p].at[pl.ds(0, CH)], sem_g)
        pltpu.async_copy(p_hbm.at[scur[p].at[pl.ds(CH, CH)]],
                         rows[p].at[pl.ds(CH, CH)], sem_g)

    def gather_wait(p):
        pltpu.make_async_copy(p_hbm.at[scur[p].at[pl.ds(0, CH)]],
                              rows[p].at[pl.ds(0, CH)], sem_g).wait()
        pltpu.make_async_copy(p_hbm.at[scur[p].at[pl.ds(CH, CH)]],
                              rows[p].at[pl.ds(CH, CH)], sem_g).wait()

    def scatter_issue():
        pltpu.async_copy(orows_v, s_sh.at[dscat_v], sem_s, add=True)

    def scatter_wait():
        pltpu.make_async_copy(orows_v, s_sh.at[dscat_v], sem_s).wait()

    dnums = lax.GatherDimensionNumbers(offset_dims=(), collapsed_slice_dims=(0,),
                                       start_index_map=(0,))

    def compute(p):
        rp = rows[p]
        ep = ecur[p]

        @plsc.parallel_loop(0, C // 16)
        def _grp(g):
            D = ep[pl.ds(g * 16, 16)]
            for j in range(0, 16, 2):
                es = (g * 16 + j, g * 16 + j + 1)
                ds = []
                for jj in (j, j + 1):
                    lane = jnp.full((16, 1), jj, jnp.int32)
                    ds.append(lax.gather(D, lane, dnums, slice_sizes=(1,),
                                         mode=lax.GatherScatterMode.PROMISE_IN_BOUNDS))
                vals = [[rp[e, pl.ds(k * 16, 16)] for k in range(H // 16)]
                        for e in es]
                outs = [[jnp.maximum(vals[t][k] + ds[t] * wd[k], 0.0)
                         for k in range(H // 16)] for t in range(2)]
                for t in range(2):
                    for k in range(H // 16):
                        orows_v[es[t], pl.ds(k * 16, 16)] = outs[t][k]

    def dscat_fill(p):
        dp = dcur[p]
        for g in range(C // 16):
            dscat_v[pl.ds(g * 16, 16)] = dp[pl.ds(g * 16, 16)]

    plsc.subcore_barrier()

    idx_issue(0, 0)
    idx_wait(0)
    gather_issue(0)
    idx_issue(1, 1)
    gather_wait(0)
    idx_wait(1)
    gather_issue(1)
    compute(0)
    dscat_fill(0)
    scatter_issue()
    idx_issue(2, 0)

    @pl.loop(0, (NCHT - 1) // 2)
    def _pair(t):
        for b in range(2):
            i = 2 * t + 1 + b
            p = 1 - b
            gather_wait(p)

            @pl.when(i + 1 < NCHT)
            def _():
                idx_wait(1 - p)
                gather_issue(1 - p)

            scatter_wait()
            compute(p)
            dscat_fill(p)
            scatter_issue()

            @pl.when(i + 2 < NCHT)
            def _():
                idx_issue(i + 2, p)

    scatter_wait()

    plsc.subcore_barrier()

    pltpu.sync_copy(s_sh.at[pl.ds(s * RPT, RPT)],
                    out_hbm.at[c, pl.ds(s * RPT, RPT)])


@functools.lru_cache(maxsize=1)
def _sc_edge_call():
    return pl.kernel(
        _sc_edges,
        out_type=jax.ShapeDtypeStruct((NC, NP, HP), jnp.float32),
        mesh=plsc.VectorSubcoreMesh(core_axis_name="c", subcore_axis_name="s",
                                    num_cores=NC, num_subcores=NS),
        compiler_params=pltpu.CompilerParams(use_tc_tiling_on_sc=False,
                                             needs_layout_passes=False),
        scratch_types=[
            pltpu.VMEM((C,), jnp.int32),
            pltpu.VMEM((C,), jnp.int32),
            pltpu.VMEM((C,), jnp.int32),
            pltpu.VMEM((C,), jnp.int32),
            pltpu.VMEM((C,), jnp.float32),
            pltpu.VMEM((C,), jnp.float32),
            pltpu.VMEM((C,), jnp.int32),
            pltpu.VMEM((C, H), jnp.float32),
            pltpu.VMEM((C, H), jnp.float32),
            pltpu.VMEM((C, HP), jnp.float32),
            pltpu.VMEM((H,), jnp.float32),
            pltpu.VMEM((16, HP), jnp.float32),
            pltpu.VMEM_SHARED((NP, HP), jnp.float32),
            pltpu.SemaphoreType.DMA,
            pltpu.SemaphoreType.DMA,
            pltpu.SemaphoreType.DMA,
        ],
    )


def _tc_pre(x_ref, w_ref, b_ref, o_ref):
    o_ref[:] = jnp.dot(x_ref[:], w_ref[:],
                       preferred_element_type=jnp.float32) + b_ref[:]


def _tc_post(x_ref, s_ref, w2, b2, w3a, w3b, b3, w4, b4, g_ref, bt_ref, o_ref):
    S = s_ref[0, :N] + s_ref[1, :N]
    hsum = S[:, :H]
    deg = S[:, H:H + 1]
    agg = jnp.dot(hsum, w2[:], preferred_element_type=jnp.float32) + deg * b2[:]
    u = jnp.maximum(
        jnp.dot(x_ref[:], w3a[:], preferred_element_type=jnp.float32)
        + jnp.dot(agg, w3b[:], preferred_element_type=jnp.float32)
        + b3[:], 0.0)
    upd = jnp.dot(u, w4[:], preferred_element_type=jnp.float32) + b4[:]
    y = x_ref[:] + upd
    mu = jnp.mean(y, axis=-1, keepdims=True)
    var = jnp.mean((y - mu) * (y - mu), axis=-1, keepdims=True)
    o_ref[:] = (y - mu) * lax.rsqrt(var + 1e-5) * g_ref[:] + bt_ref[:]


def kernel(x, edge_dir, W1, b1, W2, b2, W3, b3, W4, b4, gamma, beta,
           src_idx, dst_idx):
    W1a = W1[:H]
    wd = W1[H]
    W3a = W3[:H]
    W3b = W3[H:]

    P = pl.pallas_call(
        _tc_pre,
        out_shape=jax.ShapeDtypeStruct((N, H), jnp.float32),
    )(x, W1a, b1.reshape(1, H))

    Sp = _sc_edge_call()(
        P,
        src_idx,
        dst_idx,
        edge_dir.reshape(E),
        wd,
    )

    out = pl.pallas_call(
        _tc_post,
        out_shape=jax.ShapeDtypeStruct((N, H), jnp.float32),
    )(x, Sp, W2, b2.reshape(1, H), W3a, W3b, b3.reshape(1, H),
      W4, b4.reshape(1, H), gamma.reshape(1, H), beta.reshape(1, H))
    return out
